# baseline (speedup 1.0000x reference)
import functools

import jax
import jax.numpy as jnp
from jax import lax
from jax.experimental import pallas as pl
from jax.experimental.pallas import tpu as pltpu
from jax.experimental.pallas import tpu_sc as plsc

NC = 2
NS = 16
NW = NC * NS
LANES = 16
G = 8
NBUF = 2
CORE0_FRAC = 0.43


def _full16(v):
    return jnp.full((LANES,), v, dtype=jnp.int32)


def _sc_gather_reduce(nidx, nodes, w, feat_table, ng0, ng1, d, k):
    gk = G * k
    bpad = NS * (ng0 + ng1) * G
    dsl = d // LANES
    ngmax = max(ng0, ng1)

    mesh = plsc.VectorSubcoreMesh(core_axis_name="c", subcore_axis_name="s")

    @functools.partial(
        pl.kernel,
        mesh=mesh,
        compiler_params=pltpu.CompilerParams(needs_layout_passes=False),
        out_type=[
            jax.ShapeDtypeStruct((bpad, d), jnp.float32),
            jax.ShapeDtypeStruct((bpad, d), jnp.float32),
        ],
        scratch_types=(
            [pltpu.VMEM((ngmax * gk,), jnp.int32),
             pltpu.VMEM((ngmax * G,), jnp.int32),
             pltpu.VMEM((ngmax * gk,), jnp.float32)]
            + [pltpu.VMEM((gk, d), jnp.float32) for _ in range(NBUF)]
            + [pltpu.VMEM((G, d), jnp.float32) for _ in range(NBUF)]
            + [pltpu.VMEM((G, d), jnp.float32) for _ in range(NBUF)]
            + [pltpu.SemaphoreType.DMA for _ in range(4 * NBUF)]
        ),
    )
    def sc_kernel(nidx_hbm, nodes_hbm, w_hbm, table_hbm, neigh_hbm, self_hbm,
                  nidx_sl, nodes_sl, w_sl, *bufs):
        cid = lax.axis_index("c")
        sid = lax.axis_index("s")
        wid = sid * NC + cid
        ngc = jnp.where(cid == 0, ng0, ng1)
        row_base = (sid * (ng0 + ng1) + cid * ng0) * G
        rows_b = bufs[:NBUF]
        sst_b = bufs[NBUF:2 * NBUF]
        nout_b = bufs[2 * NBUF:3 * NBUF]
        gsem_b = bufs[3 * NBUF:4 * NBUF]
        sgsem_b = bufs[4 * NBUF:5 * NBUF]
        nsem_b = bufs[5 * NBUF:6 * NBUF]
        ssem_b = bufs[6 * NBUF:7 * NBUF]

        pltpu.sync_copy(nidx_hbm.at[wid], nidx_sl)
        pltpu.sync_copy(nodes_hbm.at[wid], nodes_sl)
        pltpu.sync_copy(w_hbm.at[wid], w_sl)

        def big_gather(g, p):
            return pltpu.make_async_copy(
                table_hbm.at[nidx_sl.at[pl.ds(g * gk, gk)]],
                rows_b[p], gsem_b[p])

        def self_gather(g, p):
            return pltpu.make_async_copy(
                table_hbm.at[nodes_sl.at[pl.ds(g * G, G)]],
                sst_b[p], sgsem_b[p])

        def out_copies(g, p):
            row0 = row_base + g * G
            nc = pltpu.make_async_copy(
                nout_b[p], neigh_hbm.at[pl.ds(row0, G), :], nsem_b[p])
            sc = pltpu.make_async_copy(
                sst_b[p], self_hbm.at[pl.ds(row0, G), :], ssem_b[p])
            return nc, sc

        for p0 in range(NBUF):
            big_gather(p0, p0).start()
            self_gather(p0, p0).start()

        def step(g, p):
            rows, nout = rows_b[p], nout_b[p]

            @pl.when(g >= NBUF)
            def _():
                nc, sc = out_copies(g - NBUF, p)
                nc.wait()
                sc.wait()
                self_gather(g, p).start()

            big_gather(g, p).wait()

            def body_b(b, _):
                base = b * k
                wbase = g * gk + base
                wv = [plsc.load_gather(w_sl, [_full16(wbase + j)])
                      for j in range(k)]
                wsum = wv[0]
                for j in range(1, k):
                    wsum = wsum + wv[j]
                inv = 1.0 / wsum
                for ds in range(dsl):
                    sl = pl.ds(ds * LANES, LANES)
                    acc = wv[0] * rows[base, sl]
                    for j in range(1, k):
                        acc = acc + wv[j] * rows[base + j, sl]
                    nout[b, sl] = acc * inv
                return 0

            lax.fori_loop(0, G, body_b, 0)

            self_gather(g, p).wait()
            nc, sc = out_copies(g, p)
            nc.start()
            sc.start()

            @pl.when(g + NBUF < ngc)
            def _():
                big_gather(g + NBUF, p).start()

        def loop_body(i, _):
            for p in range(NBUF):
                step(NBUF * i + p, p)
            return 0

        lax.fori_loop(0, ngc // NBUF, loop_body, 0)

        for p in range(NBUF):
            nc, sc = out_copies(ngc - NBUF + p, p)
            nc.wait()
            sc.wait()

    return sc_kernel(nidx, nodes, w, feat_table)


def _tc_dense(self_raw, neigh_feats, W_init, b_init, W_final, b_final,
              bm, b_rows):
    bpad, d = self_raw.shape
    e = W_init.shape[1]

    def body(x_ref, n_ref, wi_ref, wf_ref, bi_ref, bf_ref, o_ref):
        wc = jnp.dot(wi_ref[...], wf_ref[0:e, :],
                     preferred_element_type=jnp.float32)
        bias = jnp.dot(bi_ref[...], wf_ref[0:e, :],
                       preferred_element_type=jnp.float32) + bf_ref[...]
        out = (jnp.dot(x_ref[...], wc, preferred_element_type=jnp.float32)
               + jnp.dot(n_ref[...], wf_ref[e:, :],
                         preferred_element_type=jnp.float32)
               + bias)
        o_ref[...] = out * jax.nn.sigmoid(out)

    return pl.pallas_call(
        body,
        grid=(bpad // bm,),
        in_specs=[
            pl.BlockSpec((bm, d), lambda i: (i, 0)),
            pl.BlockSpec((bm, d), lambda i: (i, 0)),
            pl.BlockSpec(W_init.shape, lambda i: (0, 0)),
            pl.BlockSpec(W_final.shape, lambda i: (0, 0)),
            pl.BlockSpec((1, e), lambda i: (0, 0)),
            pl.BlockSpec((1, e), lambda i: (0, 0)),
        ],
        out_specs=pl.BlockSpec((bm, e), lambda i: (i, 0)),
        out_shape=jax.ShapeDtypeStruct((b_rows, e), jnp.float32),
    )(self_raw, neigh_feats, W_init, W_final,
      b_init.reshape(1, e), b_final.reshape(1, e))


def kernel(nodes, neigh_idx, neigh_w, feat_table, W_init, b_init,
           W_final, b_final):
    b, k = neigh_idx.shape
    d = feat_table.shape[1]

    chunk = NW * G * NBUF
    bpad = ((b + chunk - 1) // chunk) * chunk
    ng = bpad // (NW * G)
    pad = bpad - b

    ngp = 2 * ng
    ng0 = max(NBUF, (int(ngp * CORE0_FRAC) // NBUF) * NBUF)
    ng1 = ngp - ng0
    ngmax = max(ng0, ng1)

    def _split(flat, per_g, fill):
        x2 = flat.reshape(NS, ngp * per_g)
        a = jnp.pad(x2[:, :ng0 * per_g],
                    ((0, 0), (0, (ngmax - ng0) * per_g)),
                    constant_values=fill)
        b2 = jnp.pad(x2[:, ng0 * per_g:],
                     ((0, 0), (0, (ngmax - ng1) * per_g)),
                     constant_values=fill)
        return jnp.stack([a, b2], axis=1).reshape(NW, ngmax * per_g)

    nidx_f = _split(jnp.pad(neigh_idx.reshape(-1), (0, pad * k)), G * k, 0)
    w_f = _split(jnp.pad(neigh_w.reshape(-1), (0, pad * k),
                         constant_values=1.0), G * k, 1.0)
    nodes_f = _split(jnp.pad(nodes, (0, pad)), G, 0)

    neigh_feats, self_raw = _sc_gather_reduce(nidx_f, nodes_f, w_f,
                                              feat_table, ng0, ng1, d, k)
    return _tc_dense(self_raw, neigh_feats, W_init, b_init, W_final, b_final,
                     bm=1024 if bpad % 1024 == 0 else 512, b_rows=b)


# speedup vs baseline: 1.0943x; 1.0943x over previous
import functools

import jax
import jax.numpy as jnp
from jax import lax
from jax.experimental import pallas as pl
from jax.experimental.pallas import tpu as pltpu
from jax.experimental.pallas import tpu_sc as plsc

NC = 2
NS = 16
NW = NC * NS
LANES = 16
G = 8
NBUF = 2
CORE0_FRAC = 0.57


def _full16(v):
    return jnp.full((LANES,), v, dtype=jnp.int32)


def _sc_gather_reduce(nidx, nodes, w, feat_table, ng0, ng1, d, k):
    gk = G * k
    bpad = NS * (ng0 + ng1) * G
    dsl = d // LANES
    ngmax = max(ng0, ng1)

    mesh = plsc.VectorSubcoreMesh(core_axis_name="c", subcore_axis_name="s")

    @functools.partial(
        pl.kernel,
        mesh=mesh,
        compiler_params=pltpu.CompilerParams(needs_layout_passes=False),
        out_type=[
            jax.ShapeDtypeStruct((bpad, d), jnp.float32),
            jax.ShapeDtypeStruct((bpad, d), jnp.float32),
        ],
        scratch_types=(
            [pltpu.VMEM((ngmax * gk,), jnp.int32),
             pltpu.VMEM((ngmax * G,), jnp.int32),
             pltpu.VMEM((ngmax * gk,), jnp.float32)]
            + [pltpu.VMEM((gk, d), jnp.float32) for _ in range(NBUF)]
            + [pltpu.VMEM((G, d), jnp.float32) for _ in range(NBUF)]
            + [pltpu.VMEM((G, d), jnp.float32) for _ in range(NBUF)]
            + [pltpu.SemaphoreType.DMA for _ in range(4 * NBUF)]
        ),
    )
    def sc_kernel(nidx_hbm, nodes_hbm, w_hbm, table_hbm, neigh_hbm, self_hbm,
                  nidx_sl, nodes_sl, w_sl, *bufs):
        cid = lax.axis_index("c")
        sid = lax.axis_index("s")
        wid = sid * NC + cid
        ngc = jnp.where(cid == 0, ng0, ng1)
        row_base = (sid * (ng0 + ng1) + cid * ng0) * G
        rows_b = bufs[:NBUF]
        sst_b = bufs[NBUF:2 * NBUF]
        nout_b = bufs[2 * NBUF:3 * NBUF]
        gsem_b = bufs[3 * NBUF:4 * NBUF]
        sgsem_b = bufs[4 * NBUF:5 * NBUF]
        nsem_b = bufs[5 * NBUF:6 * NBUF]
        ssem_b = bufs[6 * NBUF:7 * NBUF]

        pltpu.sync_copy(nidx_hbm.at[wid], nidx_sl)
        pltpu.sync_copy(nodes_hbm.at[wid], nodes_sl)
        pltpu.sync_copy(w_hbm.at[wid], w_sl)

        def big_gather(g, p):
            return pltpu.make_async_copy(
                table_hbm.at[nidx_sl.at[pl.ds(g * gk, gk)]],
                rows_b[p], gsem_b[p])

        def self_gather(g, p):
            return pltpu.make_async_copy(
                table_hbm.at[nodes_sl.at[pl.ds(g * G, G)]],
                sst_b[p], sgsem_b[p])

        def out_copies(g, p):
            row0 = row_base + g * G
            nc = pltpu.make_async_copy(
                nout_b[p], neigh_hbm.at[pl.ds(row0, G), :], nsem_b[p])
            sc = pltpu.make_async_copy(
                sst_b[p], self_hbm.at[pl.ds(row0, G), :], ssem_b[p])
            return nc, sc

        for p0 in range(NBUF):
            big_gather(p0, p0).start()
            self_gather(p0, p0).start()

        def step(g, p):
            rows, nout = rows_b[p], nout_b[p]

            @pl.when(g >= NBUF)
            def _():
                nc, sc = out_copies(g - NBUF, p)
                nc.wait()
                sc.wait()
                self_gather(g, p).start()

            big_gather(g, p).wait()

            def body_b(b, _):
                base = b * k
                wbase = g * gk + base
                wv = [plsc.load_gather(w_sl, [_full16(wbase + j)])
                      for j in range(k)]
                wsum = wv[0]
                for j in range(1, k):
                    wsum = wsum + wv[j]
                inv = 1.0 / wsum
                for ds in range(dsl):
                    sl = pl.ds(ds * LANES, LANES)
                    acc = wv[0] * rows[base, sl]
                    for j in range(1, k):
                        acc = acc + wv[j] * rows[base + j, sl]
                    nout[b, sl] = acc * inv
                return 0

            lax.fori_loop(0, G, body_b, 0)

            self_gather(g, p).wait()
            nc, sc = out_copies(g, p)
            nc.start()
            sc.start()

            @pl.when(g + NBUF < ngc)
            def _():
                big_gather(g + NBUF, p).start()

        def loop_body(i, _):
            for p in range(NBUF):
                step(NBUF * i + p, p)
            return 0

        lax.fori_loop(0, ngc // NBUF, loop_body, 0)

        for p in range(NBUF):
            nc, sc = out_copies(ngc - NBUF + p, p)
            nc.wait()
            sc.wait()

    return sc_kernel(nidx, nodes, w, feat_table)


def _tc_dense(self_raw, neigh_feats, W_init, b_init, W_final, b_final,
              bm, b_rows):
    bpad, d = self_raw.shape
    e = W_init.shape[1]

    def body(x_ref, n_ref, wi_ref, wf_ref, bi_ref, bf_ref, o_ref):
        wc = jnp.dot(wi_ref[...], wf_ref[0:e, :],
                     preferred_element_type=jnp.float32)
        bias = jnp.dot(bi_ref[...], wf_ref[0:e, :],
                       preferred_element_type=jnp.float32) + bf_ref[...]
        out = (jnp.dot(x_ref[...], wc, preferred_element_type=jnp.float32)
               + jnp.dot(n_ref[...], wf_ref[e:, :],
                         preferred_element_type=jnp.float32)
               + bias)
        o_ref[...] = out * jax.nn.sigmoid(out)

    return pl.pallas_call(
        body,
        grid=(bpad // bm,),
        in_specs=[
            pl.BlockSpec((bm, d), lambda i: (i, 0)),
            pl.BlockSpec((bm, d), lambda i: (i, 0)),
            pl.BlockSpec(W_init.shape, lambda i: (0, 0)),
            pl.BlockSpec(W_final.shape, lambda i: (0, 0)),
            pl.BlockSpec((1, e), lambda i: (0, 0)),
            pl.BlockSpec((1, e), lambda i: (0, 0)),
        ],
        out_specs=pl.BlockSpec((bm, e), lambda i: (i, 0)),
        out_shape=jax.ShapeDtypeStruct((b_rows, e), jnp.float32),
    )(self_raw, neigh_feats, W_init, W_final,
      b_init.reshape(1, e), b_final.reshape(1, e))


def kernel(nodes, neigh_idx, neigh_w, feat_table, W_init, b_init,
           W_final, b_final):
    b, k = neigh_idx.shape
    d = feat_table.shape[1]

    chunk = NW * G * NBUF
    bpad = ((b + chunk - 1) // chunk) * chunk
    ng = bpad // (NW * G)
    pad = bpad - b

    ngp = 2 * ng
    ng0 = max(NBUF, (int(ngp * CORE0_FRAC) // NBUF) * NBUF)
    ng1 = ngp - ng0
    ngmax = max(ng0, ng1)

    def _split(flat, per_g, fill):
        x2 = flat.reshape(NS, ngp * per_g)
        a = jnp.pad(x2[:, :ng0 * per_g],
                    ((0, 0), (0, (ngmax - ng0) * per_g)),
                    constant_values=fill)
        b2 = jnp.pad(x2[:, ng0 * per_g:],
                     ((0, 0), (0, (ngmax - ng1) * per_g)),
                     constant_values=fill)
        return jnp.stack([a, b2], axis=1).reshape(NW, ngmax * per_g)

    nidx_f = _split(jnp.pad(neigh_idx.reshape(-1), (0, pad * k)), G * k, 0)
    w_f = _split(jnp.pad(neigh_w.reshape(-1), (0, pad * k),
                         constant_values=1.0), G * k, 1.0)
    nodes_f = _split(jnp.pad(nodes, (0, pad)), G, 0)

    neigh_feats, self_raw = _sc_gather_reduce(nidx_f, nodes_f, w_f,
                                              feat_table, ng0, ng1, d, k)
    return _tc_dense(self_raw, neigh_feats, W_init, b_init, W_final, b_final,
                     bm=1024 if bpad % 1024 == 0 else 512, b_rows=b)


# speedup vs baseline: 1.1422x; 1.0437x over previous
import functools

import jax
import jax.numpy as jnp
from jax import lax
from jax.experimental import pallas as pl
from jax.experimental.pallas import tpu as pltpu
from jax.experimental.pallas import tpu_sc as plsc

NC = 2
NS = 16
NW = NC * NS
LANES = 16
G = 8
NBUF = 2
CORE0_FRAC = 0.57


def _full16(v):
    return jnp.full((LANES,), v, dtype=jnp.int32)


def _sc_gather_reduce(nidx, nodes, w, feat_table, ng0, ng1, d, k):
    gk = G * k
    bpad = NS * (ng0 + ng1) * G
    dsl = d // LANES
    ngmax = max(ng0, ng1)

    mesh = plsc.VectorSubcoreMesh(core_axis_name="c", subcore_axis_name="s")

    @functools.partial(
        pl.kernel,
        mesh=mesh,
        compiler_params=pltpu.CompilerParams(needs_layout_passes=False),
        out_type=[
            jax.ShapeDtypeStruct((bpad, d), jnp.float32),
            jax.ShapeDtypeStruct((bpad, d), jnp.float32),
        ],
        scratch_types=(
            [pltpu.VMEM((ngmax * gk,), jnp.int32),
             pltpu.VMEM((ngmax * G,), jnp.int32),
             pltpu.VMEM((ngmax * gk,), jnp.float32)]
            + [pltpu.VMEM((gk, d), jnp.float32) for _ in range(NBUF)]
            + [pltpu.VMEM((G, d), jnp.float32) for _ in range(NBUF)]
            + [pltpu.VMEM((G, d), jnp.float32) for _ in range(NBUF)]
            + [pltpu.SemaphoreType.DMA for _ in range(4 * NBUF)]
        ),
    )
    def sc_kernel(nidx_hbm, nodes_hbm, w_hbm, table_hbm, neigh_hbm, self_hbm,
                  nidx_sl, nodes_sl, w_sl, *bufs):
        cid = lax.axis_index("c")
        sid = lax.axis_index("s")
        ngc = jnp.where(cid == 0, ng0, ng1)
        row_base = (sid * (ng0 + ng1) + cid * ng0) * G
        rows_b = bufs[:NBUF]
        sst_b = bufs[NBUF:2 * NBUF]
        nout_b = bufs[2 * NBUF:3 * NBUF]
        gsem_b = bufs[3 * NBUF:4 * NBUF]
        sgsem_b = bufs[4 * NBUF:5 * NBUF]
        nsem_b = bufs[5 * NBUF:6 * NBUF]
        ssem_b = bufs[6 * NBUF:7 * NBUF]

        pltpu.sync_copy(nidx_hbm.at[pl.ds(row_base * k, ngmax * gk)], nidx_sl)
        pltpu.sync_copy(nodes_hbm.at[pl.ds(row_base, ngmax * G)], nodes_sl)
        pltpu.sync_copy(w_hbm.at[pl.ds(row_base * k, ngmax * gk)], w_sl)

        def big_gather(g, p):
            return pltpu.make_async_copy(
                table_hbm.at[nidx_sl.at[pl.ds(g * gk, gk)]],
                rows_b[p], gsem_b[p])

        def self_gather(g, p):
            return pltpu.make_async_copy(
                table_hbm.at[nodes_sl.at[pl.ds(g * G, G)]],
                sst_b[p], sgsem_b[p])

        def out_copies(g, p):
            row0 = row_base + g * G
            nc = pltpu.make_async_copy(
                nout_b[p], neigh_hbm.at[pl.ds(row0, G), :], nsem_b[p])
            sc = pltpu.make_async_copy(
                sst_b[p], self_hbm.at[pl.ds(row0, G), :], ssem_b[p])
            return nc, sc

        for p0 in range(NBUF):
            big_gather(p0, p0).start()
            self_gather(p0, p0).start()

        def step(g, p):
            rows, nout = rows_b[p], nout_b[p]

            @pl.when(g >= NBUF)
            def _():
                nc, sc = out_copies(g - NBUF, p)
                nc.wait()
                sc.wait()
                self_gather(g, p).start()

            big_gather(g, p).wait()

            def body_b(b, _):
                base = b * k
                wbase = g * gk + base
                wv = [plsc.load_gather(w_sl, [_full16(wbase + j)])
                      for j in range(k)]
                wsum = wv[0]
                for j in range(1, k):
                    wsum = wsum + wv[j]
                inv = 1.0 / wsum
                for ds in range(dsl):
                    sl = pl.ds(ds * LANES, LANES)
                    acc = wv[0] * rows[base, sl]
                    for j in range(1, k):
                        acc = acc + wv[j] * rows[base + j, sl]
                    nout[b, sl] = acc * inv
                return 0

            lax.fori_loop(0, G, body_b, 0)

            self_gather(g, p).wait()
            nc, sc = out_copies(g, p)
            nc.start()
            sc.start()

            @pl.when(g + NBUF < ngc)
            def _():
                big_gather(g + NBUF, p).start()

        def loop_body(i, _):
            for p in range(NBUF):
                step(NBUF * i + p, p)
            return 0

        lax.fori_loop(0, ngc // NBUF, loop_body, 0)

        for p in range(NBUF):
            nc, sc = out_copies(ngc - NBUF + p, p)
            nc.wait()
            sc.wait()

    return sc_kernel(nidx, nodes, w, feat_table)


def _tc_dense(self_raw, neigh_feats, W_init, b_init, W_final, b_final,
              bm, b_rows):
    bpad, d = self_raw.shape
    e = W_init.shape[1]

    def body(x_ref, n_ref, wi_ref, wf_ref, bi_ref, bf_ref, o_ref):
        wc = jnp.dot(wi_ref[...], wf_ref[0:e, :],
                     preferred_element_type=jnp.float32)
        bias = jnp.dot(bi_ref[...], wf_ref[0:e, :],
                       preferred_element_type=jnp.float32) + bf_ref[...]
        out = (jnp.dot(x_ref[...], wc, preferred_element_type=jnp.float32)
               + jnp.dot(n_ref[...], wf_ref[e:, :],
                         preferred_element_type=jnp.float32)
               + bias)
        o_ref[...] = out * jax.nn.sigmoid(out)

    return pl.pallas_call(
        body,
        grid=(bpad // bm,),
        in_specs=[
            pl.BlockSpec((bm, d), lambda i: (i, 0)),
            pl.BlockSpec((bm, d), lambda i: (i, 0)),
            pl.BlockSpec(W_init.shape, lambda i: (0, 0)),
            pl.BlockSpec(W_final.shape, lambda i: (0, 0)),
            pl.BlockSpec((1, e), lambda i: (0, 0)),
            pl.BlockSpec((1, e), lambda i: (0, 0)),
        ],
        out_specs=pl.BlockSpec((bm, e), lambda i: (i, 0)),
        out_shape=jax.ShapeDtypeStruct((b_rows, e), jnp.float32),
    )(self_raw, neigh_feats, W_init, W_final,
      b_init.reshape(1, e), b_final.reshape(1, e))


def kernel(nodes, neigh_idx, neigh_w, feat_table, W_init, b_init,
           W_final, b_final):
    b, k = neigh_idx.shape
    d = feat_table.shape[1]

    chunk = NW * G * NBUF
    bpad = ((b + chunk - 1) // chunk) * chunk
    ng = bpad // (NW * G)
    pad = bpad - b

    ngp = 2 * ng
    ng0 = max(NBUF, (int(ngp * CORE0_FRAC) // NBUF) * NBUF)
    ng1 = ngp - ng0
    ngmax = max(ng0, ng1)

    over = (ngmax - min(ng0, ng1)) * G
    nidx_f = jnp.pad(neigh_idx.reshape(-1), (0, (pad + over) * k))
    w_f = jnp.pad(neigh_w.reshape(-1), (0, (pad + over) * k),
                  constant_values=1.0)
    nodes_f = jnp.pad(nodes, (0, pad + over))

    neigh_feats, self_raw = _sc_gather_reduce(nidx_f, nodes_f, w_f,
                                              feat_table, ng0, ng1, d, k)
    return _tc_dense(self_raw, neigh_feats, W_init, b_init, W_final, b_final,
                     bm=1024 if bpad % 1024 == 0 else 512, b_rows=b)
